# baseline (speedup 1.0000x reference)
import functools

import jax
import jax.numpy as jnp
from jax import lax
from jax.experimental import pallas as pl
from jax.experimental.pallas import tpu as pltpu
from jax.experimental.pallas import tpu_sc as plsc

VOCAB = 1000000
HIDDEN = 64
SEQ = 200
BATCH = 4096
EPS = 1e-12

L = 16
NV = HIDDEN // L
NTOK = BATCH * SEQ
NW = 32
PER_W = NTOK // NW
C = 256
HF = 128
NCHUNK = PER_W // C
R = NCHUNK // 2
UNROLL = 16
OPITCH = 257


def _rsqrt_v(x):
    i = lax.bitcast_convert_type(x, jnp.int32)
    i = jnp.int32(0x5F3759DF) - lax.shift_right_arithmetic(i, 1)
    y = lax.bitcast_convert_type(i, jnp.float32)
    for _ in range(2):
        y = y * (jnp.float32(1.5) - jnp.float32(0.5) * x * y * y)
    return y


@functools.partial(
    pl.kernel,
    mesh=plsc.VectorSubcoreMesh(core_axis_name="c", subcore_axis_name="s"),
    out_type=jax.ShapeDtypeStruct(
        (SEQ, HIDDEN // 8, BATCH // 128, 8, 128), jnp.float32),
    scratch_types=[
        pltpu.VMEM((2, HF), jnp.int32),
        pltpu.VMEM((2, HF), jnp.int32),
        pltpu.VMEM((C, HIDDEN), jnp.float32),
        pltpu.VMEM((C, HIDDEN), jnp.float32),
        pltpu.VMEM((HIDDEN, OPITCH), jnp.float32),
        pltpu.VMEM((HIDDEN, OPITCH), jnp.float32),
        pltpu.VMEM((SEQ, HIDDEN), jnp.float32),
        pltpu.VMEM((HIDDEN,), jnp.float32),
        pltpu.VMEM((HIDDEN,), jnp.float32),
        pltpu.SemaphoreType.DMA,
        pltpu.SemaphoreType.DMA,
        pltpu.SemaphoreType.DMA,
        pltpu.SemaphoreType.DMA,
        pltpu.SemaphoreType.DMA,
        pltpu.SemaphoreType.DMA,
    ],
    compiler_params=pltpu.CompilerParams(
        use_tc_tiling_on_sc=False, needs_layout_passes=False),
)
def _emb_ln(ids_hbm, tok_hbm, pos_hbm, gam_hbm, bet_hbm, out_hbm,
            idx0, idx1, gb0, gb1, ob0, ob1, pos_v, gam_v, bet_v,
            sg0, sg1, si0, si1, so0, so1):
    cc = lax.axis_index("c")
    ss = lax.axis_index("s")
    base_w = (ss * 2 + cc) * PER_W

    idx = (idx0, idx1)
    gb = (gb0, gb1)
    ob = (ob0, ob1)
    sg = (sg0, sg1)
    si = (si0, si1)
    so = (so0, so1)

    io = lax.iota(jnp.int32, L)
    perms = [lax.bitwise_xor(io, jnp.int32(d)) for d in (8, 4, 2, 1)]
    hidx = [io + jnp.int32(L * k) for k in range(NV)]

    def crosslane_sum(x):
        for pm in perms:
            x = x + x.at[pm].get(mode="promise_in_bounds")
        return x

    pltpu.sync_copy(pos_hbm.at[pl.ds(0, SEQ)], pos_v)
    pltpu.sync_copy(gam_hbm, gam_v)
    pltpu.sync_copy(bet_hbm, bet_v)

    def fire_idx(ch, q, sem):
        for h in range(2):
            pltpu.async_copy(
                ids_hbm.at[pl.ds(base_w + ch * C + h * HF, HF)],
                idx[q].at[h], sem)

    def wait_idx(q, sem):
        for h in range(2):
            pltpu.make_async_copy(
                ids_hbm.at[pl.ds(0, HF)], idx[q].at[h], sem).wait()

    def fire_gather(q, sem):
        for h in range(2):
            pltpu.async_copy(
                tok_hbm.at[idx[q].at[h]],
                gb[q].at[pl.ds(h * HF, HF)], sem)

    def wait_gather(q, sem):
        for h in range(2):
            pltpu.make_async_copy(
                tok_hbm.at[idx[q].at[h]],
                gb[q].at[pl.ds(h * HF, HF)], sem).wait()

    def fire_wb(ch, q, sem):
        flat = base_w + ch * C
        s = lax.shift_right_logical(flat, 12)
        btg = lax.shift_right_logical(flat & jnp.int32(4095), 7)
        for ht in range(HIDDEN // 8):
            for bt in range(C // 128):
                pltpu.async_copy(
                    ob[q].at[pl.ds(ht * 8, 8), pl.ds(bt * 128, 128)],
                    out_hbm.at[s, ht, btg + bt], sem)

    def wait_wb(q, sem):
        for ht in range(HIDDEN // 8):
            for bt in range(C // 128):
                pltpu.make_async_copy(
                    ob[q].at[pl.ds(ht * 8, 8), pl.ds(bt * 128, 128)],
                    out_hbm.at[0, 0, 0], sem).wait()

    def compute(ch, q):
        p = lax.shift_right_logical(base_w + ch * C, 12)
        pvs = [pos_v[p, pl.ds(L * k, L)] for k in range(NV)]
        gvs = [gam_v[pl.ds(L * k, L)] for k in range(NV)]
        bvs = [bet_v[pl.ds(L * k, L)] for k in range(NV)]
        gq = gb[q]
        oq = ob[q]

        @plsc.parallel_loop(0, C, unroll=UNROLL)
        def _row(i):
            xs = [gq[i, pl.ds(L * k, L)] + pvs[k] for k in range(NV)]
            tsum = (xs[0] + xs[1]) + (xs[2] + xs[3])
            qsum = (xs[0] * xs[0] + xs[1] * xs[1]) + (
                xs[2] * xs[2] + xs[3] * xs[3])
            meanv = crosslane_sum(tsum) * jnp.float32(1.0 / HIDDEN)
            m2 = crosslane_sum(qsum) * jnp.float32(1.0 / HIDDEN)
            var = m2 - meanv * meanv
            inv = _rsqrt_v(var + jnp.float32(EPS))
            bv = jnp.broadcast_to(i, (L,)).astype(jnp.int32)
            for k in range(NV):
                y = (xs[k] - meanv) * inv * gvs[k] + bvs[k]
                plsc.store_scatter(oq, [hidx[k], bv], y)

    for h in range(2):
        pltpu.sync_copy(ids_hbm.at[pl.ds(base_w + h * HF, HF)], idx[0].at[h])
    fire_gather(0, sg[0])
    fire_idx(1, 1, si[1])

    def iter_body(r, _):
        for q in range(2):
            ch = 2 * r + q
            if q == 0:
                wait_idx(1, si[1])
                fire_gather(1, sg[1])
            else:
                @pl.when(r < R - 1)
                def _():
                    wait_idx(0, si[0])
                    fire_gather(0, sg[0])
            wait_gather(q, sg[q])
            @pl.when(r < R - 1)
            def _():
                fire_idx(ch + 2, q, si[q])
            @pl.when(r > 0)
            def _():
                wait_wb(q, so[q])
            compute(ch, q)
            fire_wb(ch, q, so[q])
        return 0

    lax.fori_loop(0, R, iter_body, 0)
    wait_wb(0, so[0])
    wait_wb(1, so[1])


def kernel(input_ids, token_table, pos_table, ln_gamma, ln_beta):
    ids_t = jnp.transpose(input_ids).reshape(-1)
    out5 = _emb_ln(ids_t, token_table, pos_table, ln_gamma, ln_beta)
    return out5.transpose((2, 4, 0, 1, 3)).reshape(BATCH, SEQ, HIDDEN)


# speedup vs baseline: 1.1564x; 1.1564x over previous
import functools

import jax
import jax.numpy as jnp
from jax import lax
from jax.experimental import pallas as pl
from jax.experimental.pallas import tpu as pltpu
from jax.experimental.pallas import tpu_sc as plsc

VOCAB = 1000000
HIDDEN = 64
SEQ = 200
BATCH = 4096
EPS = 1e-12

L = 16
NV = HIDDEN // L
NTOK = BATCH * SEQ
NW = 32
PER_W = NTOK // NW
C = 256
HF = 128
NCHUNK = PER_W // C
R = NCHUNK // 2
UNROLL = 4
OPITCH = 257


def _rsqrt_v(x):
    i = lax.bitcast_convert_type(x, jnp.int32)
    i = jnp.int32(0x5F3759DF) - lax.shift_right_arithmetic(i, 1)
    y = lax.bitcast_convert_type(i, jnp.float32)
    for _ in range(2):
        y = y * (jnp.float32(1.5) - jnp.float32(0.5) * x * y * y)
    return y


@functools.partial(
    pl.kernel,
    mesh=plsc.VectorSubcoreMesh(core_axis_name="c", subcore_axis_name="s"),
    out_type=jax.ShapeDtypeStruct(
        (SEQ, HIDDEN // 8, BATCH // 128, 8, 128), jnp.float32),
    scratch_types=[
        pltpu.VMEM((2, HF), jnp.int32),
        pltpu.VMEM((2, HF), jnp.int32),
        pltpu.VMEM((C, HIDDEN), jnp.float32),
        pltpu.VMEM((C, HIDDEN), jnp.float32),
        pltpu.VMEM((HIDDEN, OPITCH), jnp.float32),
        pltpu.VMEM((HIDDEN, OPITCH), jnp.float32),
        pltpu.VMEM((SEQ, HIDDEN), jnp.float32),
        pltpu.VMEM((HIDDEN,), jnp.float32),
        pltpu.VMEM((HIDDEN,), jnp.float32),
        pltpu.SemaphoreType.DMA,
        pltpu.SemaphoreType.DMA,
        pltpu.SemaphoreType.DMA,
        pltpu.SemaphoreType.DMA,
        pltpu.SemaphoreType.DMA,
        pltpu.SemaphoreType.DMA,
    ],
    compiler_params=pltpu.CompilerParams(
        use_tc_tiling_on_sc=False, needs_layout_passes=False),
)
def _emb_ln(ids_hbm, tok_hbm, pos_hbm, gam_hbm, bet_hbm, out_hbm,
            idx0, idx1, gb0, gb1, ob0, ob1, pos_v, gam_v, bet_v,
            sg0, sg1, si0, si1, so0, so1):
    cc = lax.axis_index("c")
    ss = lax.axis_index("s")
    base_w = (ss * 2 + cc) * PER_W

    idx = (idx0, idx1)
    gb = (gb0, gb1)
    ob = (ob0, ob1)
    sg = (sg0, sg1)
    si = (si0, si1)
    so = (so0, so1)

    io = lax.iota(jnp.int32, L)
    perms = [lax.bitwise_xor(io, jnp.int32(d)) for d in (8, 4, 2, 1)]
    hidx = [io + jnp.int32(L * k) for k in range(NV)]

    def crosslane_sum(x):
        for pm in perms:
            x = x + x.at[pm].get(mode="promise_in_bounds")
        return x

    pltpu.sync_copy(pos_hbm.at[pl.ds(0, SEQ)], pos_v)
    pltpu.sync_copy(gam_hbm, gam_v)
    pltpu.sync_copy(bet_hbm, bet_v)

    def fire_idx(ch, q, sem):
        for h in range(2):
            pltpu.async_copy(
                ids_hbm.at[pl.ds(base_w + ch * C + h * HF, HF)],
                idx[q].at[h], sem)

    def wait_idx(q, sem):
        for h in range(2):
            pltpu.make_async_copy(
                ids_hbm.at[pl.ds(0, HF)], idx[q].at[h], sem).wait()

    def fire_gather(q, sem):
        for h in range(2):
            pltpu.async_copy(
                tok_hbm.at[idx[q].at[h]],
                gb[q].at[pl.ds(h * HF, HF)], sem)

    def wait_gather(q, sem):
        for h in range(2):
            pltpu.make_async_copy(
                tok_hbm.at[idx[q].at[h]],
                gb[q].at[pl.ds(h * HF, HF)], sem).wait()

    def fire_wb(ch, q, sem):
        flat = base_w + ch * C
        s = lax.shift_right_logical(flat, 12)
        btg = lax.shift_right_logical(flat & jnp.int32(4095), 7)
        for ht in range(HIDDEN // 8):
            for bt in range(C // 128):
                pltpu.async_copy(
                    ob[q].at[pl.ds(ht * 8, 8), pl.ds(bt * 128, 128)],
                    out_hbm.at[s, ht, btg + bt], sem)

    def wait_wb(q, sem):
        for ht in range(HIDDEN // 8):
            for bt in range(C // 128):
                pltpu.make_async_copy(
                    ob[q].at[pl.ds(ht * 8, 8), pl.ds(bt * 128, 128)],
                    out_hbm.at[0, 0, 0], sem).wait()

    def compute(ch, q):
        p = lax.shift_right_logical(base_w + ch * C, 12)
        pvs = [pos_v[p, pl.ds(L * k, L)] for k in range(NV)]
        gvs = [gam_v[pl.ds(L * k, L)] for k in range(NV)]
        bvs = [bet_v[pl.ds(L * k, L)] for k in range(NV)]
        gq = gb[q]
        oq = ob[q]

        @plsc.parallel_loop(0, C, unroll=UNROLL)
        def _row(i):
            xs = [gq[i, pl.ds(L * k, L)] + pvs[k] for k in range(NV)]
            tsum = (xs[0] + xs[1]) + (xs[2] + xs[3])
            qsum = (xs[0] * xs[0] + xs[1] * xs[1]) + (
                xs[2] * xs[2] + xs[3] * xs[3])
            meanv = crosslane_sum(tsum) * jnp.float32(1.0 / HIDDEN)
            m2 = crosslane_sum(qsum) * jnp.float32(1.0 / HIDDEN)
            var = m2 - meanv * meanv
            inv = _rsqrt_v(var + jnp.float32(EPS))
            bv = jnp.broadcast_to(i, (L,)).astype(jnp.int32)
            for k in range(NV):
                y = (xs[k] - meanv) * inv * gvs[k] + bvs[k]
                plsc.store_scatter(oq, [hidx[k], bv], y)

    for h in range(2):
        pltpu.sync_copy(ids_hbm.at[pl.ds(base_w + h * HF, HF)], idx[0].at[h])
    fire_gather(0, sg[0])
    fire_idx(1, 1, si[1])

    def iter_body(r, _):
        for q in range(2):
            ch = 2 * r + q
            if q == 0:
                wait_idx(1, si[1])
                fire_gather(1, sg[1])
            else:
                @pl.when(r < R - 1)
                def _():
                    wait_idx(0, si[0])
                    fire_gather(0, sg[0])
            wait_gather(q, sg[q])
            @pl.when(r < R - 1)
            def _():
                fire_idx(ch + 2, q, si[q])
            @pl.when(r > 0)
            def _():
                wait_wb(q, so[q])
            compute(ch, q)
            fire_wb(ch, q, so[q])
        return 0

    lax.fori_loop(0, R, iter_body, 0)
    wait_wb(0, so[0])
    wait_wb(1, so[1])


def kernel(input_ids, token_table, pos_table, ln_gamma, ln_beta):
    ids_t = jnp.transpose(input_ids).reshape(-1)
    out5 = _emb_ln(ids_t, token_table, pos_table, ln_gamma, ln_beta)
    return out5.transpose((2, 4, 0, 1, 3)).reshape(BATCH, SEQ, HIDDEN)


# speedup vs baseline: 1.2047x; 1.0418x over previous
import functools

import jax
import jax.numpy as jnp
from jax import lax
from jax.experimental import pallas as pl
from jax.experimental.pallas import tpu as pltpu
from jax.experimental.pallas import tpu_sc as plsc

VOCAB = 1000000
HIDDEN = 64
SEQ = 200
BATCH = 4096
EPS = 1e-12

L = 16
NV = HIDDEN // L
NTOK = BATCH * SEQ
NW = 32
PER_W = NTOK // NW
C = 256
HF = 128
NCHUNK = PER_W // C
R = NCHUNK // 2
UNROLL = 2
OPITCH = 257


def _rsqrt_v(x):
    i = lax.bitcast_convert_type(x, jnp.int32)
    i = jnp.int32(0x5F3759DF) - lax.shift_right_arithmetic(i, 1)
    y = lax.bitcast_convert_type(i, jnp.float32)
    for _ in range(2):
        y = y * (jnp.float32(1.5) - jnp.float32(0.5) * x * y * y)
    return y


@functools.partial(
    pl.kernel,
    mesh=plsc.VectorSubcoreMesh(core_axis_name="c", subcore_axis_name="s"),
    out_type=jax.ShapeDtypeStruct(
        (SEQ, HIDDEN // 8, BATCH // 128, 8, 128), jnp.float32),
    scratch_types=[
        pltpu.VMEM((2, HF), jnp.int32),
        pltpu.VMEM((2, HF), jnp.int32),
        pltpu.VMEM((C, HIDDEN), jnp.float32),
        pltpu.VMEM((C, HIDDEN), jnp.float32),
        pltpu.VMEM((HIDDEN, OPITCH), jnp.float32),
        pltpu.VMEM((HIDDEN, OPITCH), jnp.float32),
        pltpu.VMEM((SEQ, HIDDEN), jnp.float32),
        pltpu.VMEM((HIDDEN,), jnp.float32),
        pltpu.VMEM((HIDDEN,), jnp.float32),
        pltpu.SemaphoreType.DMA,
        pltpu.SemaphoreType.DMA,
        pltpu.SemaphoreType.DMA,
        pltpu.SemaphoreType.DMA,
        pltpu.SemaphoreType.DMA,
        pltpu.SemaphoreType.DMA,
    ],
    compiler_params=pltpu.CompilerParams(
        use_tc_tiling_on_sc=False, needs_layout_passes=False),
)
def _emb_ln(ids_hbm, tok_hbm, pos_hbm, gam_hbm, bet_hbm, out_hbm,
            idx0, idx1, gb0, gb1, ob0, ob1, pos_v, gam_v, bet_v,
            sg0, sg1, si0, si1, so0, so1):
    cc = lax.axis_index("c")
    ss = lax.axis_index("s")
    base_w = (ss * 2 + cc) * PER_W

    idx = (idx0, idx1)
    gb = (gb0, gb1)
    ob = (ob0, ob1)
    sg = (sg0, sg1)
    si = (si0, si1)
    so = (so0, so1)

    io = lax.iota(jnp.int32, L)
    perms = [lax.bitwise_xor(io, jnp.int32(d)) for d in (8, 4, 2, 1)]
    hidx = [io + jnp.int32(L * k) for k in range(NV)]

    def crosslane_sum(x):
        for pm in perms:
            x = x + x.at[pm].get(mode="promise_in_bounds")
        return x

    pltpu.sync_copy(pos_hbm.at[pl.ds(0, SEQ)], pos_v)
    pltpu.sync_copy(gam_hbm, gam_v)
    pltpu.sync_copy(bet_hbm, bet_v)

    def fire_idx(ch, q, sem):
        for h in range(2):
            pltpu.async_copy(
                ids_hbm.at[pl.ds(base_w + ch * C + h * HF, HF)],
                idx[q].at[h], sem)

    def wait_idx(q, sem):
        for h in range(2):
            pltpu.make_async_copy(
                ids_hbm.at[pl.ds(0, HF)], idx[q].at[h], sem).wait()

    def fire_gather(q, sem):
        for h in range(2):
            pltpu.async_copy(
                tok_hbm.at[idx[q].at[h]],
                gb[q].at[pl.ds(h * HF, HF)], sem)

    def wait_gather(q, sem):
        for h in range(2):
            pltpu.make_async_copy(
                tok_hbm.at[idx[q].at[h]],
                gb[q].at[pl.ds(h * HF, HF)], sem).wait()

    def fire_wb(ch, q, sem):
        flat = base_w + ch * C
        s = lax.shift_right_logical(flat, 12)
        btg = lax.shift_right_logical(flat & jnp.int32(4095), 7)
        for ht in range(HIDDEN // 8):
            for bt in range(C // 128):
                pltpu.async_copy(
                    ob[q].at[pl.ds(ht * 8, 8), pl.ds(bt * 128, 128)],
                    out_hbm.at[s, ht, btg + bt], sem)

    def wait_wb(q, sem):
        for ht in range(HIDDEN // 8):
            for bt in range(C // 128):
                pltpu.make_async_copy(
                    ob[q].at[pl.ds(ht * 8, 8), pl.ds(bt * 128, 128)],
                    out_hbm.at[0, 0, 0], sem).wait()

    def compute(ch, q):
        p = lax.shift_right_logical(base_w + ch * C, 12)
        pvs = [pos_v[p, pl.ds(L * k, L)] for k in range(NV)]
        gvs = [gam_v[pl.ds(L * k, L)] for k in range(NV)]
        bvs = [bet_v[pl.ds(L * k, L)] for k in range(NV)]
        gq = gb[q]
        oq = ob[q]

        @plsc.parallel_loop(0, C, unroll=UNROLL)
        def _row(i):
            xs = [gq[i, pl.ds(L * k, L)] + pvs[k] for k in range(NV)]
            tsum = (xs[0] + xs[1]) + (xs[2] + xs[3])
            qsum = (xs[0] * xs[0] + xs[1] * xs[1]) + (
                xs[2] * xs[2] + xs[3] * xs[3])
            meanv = crosslane_sum(tsum) * jnp.float32(1.0 / HIDDEN)
            m2 = crosslane_sum(qsum) * jnp.float32(1.0 / HIDDEN)
            var = m2 - meanv * meanv
            inv = _rsqrt_v(var + jnp.float32(EPS))
            bv = jnp.broadcast_to(i, (L,)).astype(jnp.int32)
            for k in range(NV):
                y = (xs[k] - meanv) * inv * gvs[k] + bvs[k]
                plsc.store_scatter(oq, [hidx[k], bv], y)

    for h in range(2):
        pltpu.sync_copy(ids_hbm.at[pl.ds(base_w + h * HF, HF)], idx[0].at[h])
    fire_gather(0, sg[0])
    fire_idx(1, 1, si[1])

    def iter_body(r, _):
        for q in range(2):
            ch = 2 * r + q
            if q == 0:
                wait_idx(1, si[1])
                fire_gather(1, sg[1])
            else:
                @pl.when(r < R - 1)
                def _():
                    wait_idx(0, si[0])
                    fire_gather(0, sg[0])
            wait_gather(q, sg[q])
            @pl.when(r < R - 1)
            def _():
                fire_idx(ch + 2, q, si[q])
            @pl.when(r > 0)
            def _():
                wait_wb(q, so[q])
            compute(ch, q)
            fire_wb(ch, q, so[q])
        return 0

    lax.fori_loop(0, R, iter_body, 0)
    wait_wb(0, so[0])
    wait_wb(1, so[1])


def kernel(input_ids, token_table, pos_table, ln_gamma, ln_beta):
    ids_t = jnp.transpose(input_ids).reshape(-1)
    out5 = _emb_ln(ids_t, token_table, pos_table, ln_gamma, ln_beta)
    return out5.transpose((2, 4, 0, 1, 3)).reshape(BATCH, SEQ, HIDDEN)
